# baseline (speedup 1.0000x reference)
import dataclasses
import functools

import jax
import jax.numpy as jnp
from jax import lax
from jax.experimental import pallas as pl
from jax.experimental.pallas import tpu as pltpu
from jax.experimental.pallas import tpu_sc as plsc

BATCH = 16384
DIM = 128
LANES = 16
NUM_WORKERS = 32
ROWS_PER_WORKER = BATCH // NUM_WORKERS
CHUNK = 16
NUM_CHUNKS = ROWS_PER_WORKER // CHUNK
GROUPS = CHUNK // LANES
NSETS = 4
NARR = 12
MARGIN = 1.0


def _make_sc_kernel():
    mesh = plsc.VectorSubcoreMesh(core_axis_name="c", subcore_axis_name="s")
    cp = pltpu.CompilerParams()
    if "needs_layout_passes" in pltpu.CompilerParams.__dataclass_fields__:
        cp = dataclasses.replace(cp, needs_layout_passes=False)

    @functools.partial(
        pl.kernel,
        out_type=jax.ShapeDtypeStruct((NUM_WORKERS, LANES), jnp.float32),
        mesh=mesh,
        compiler_params=cp,
        scratch_types=(
            [pltpu.VMEM((ROWS_PER_WORKER,), jnp.int32)] * 6
            + [pltpu.VMEM((NSETS, NARR, CHUNK, DIM), jnp.float32)]
            + [pltpu.VMEM((LANES,), jnp.float32)]
            + [pltpu.SemaphoreType.DMA] * (NSETS + 1)
        ),
    )
    def transd_sc(ent_e, ent_t, rel_e, rel_t,
                  ph, pt, pr, nh, nt, nr,
                  out,
                  iph, ipt, ipr, inh, int_, inr,
                  data, lacc_v, isem, gsem0, gsem1, gsem2, gsem3):
        wid = lax.axis_index("s") * 2 + lax.axis_index("c")
        base0 = wid * ROWS_PER_WORKER

        idx_bufs = [iph, ipt, ipr, inh, int_, inr]
        loads = []
        for src, dst in zip([ph, pt, pr, nh, nt, nr], idx_bufs):
            loads.append(pltpu.async_copy(
                src.at[pl.ds(base0, ROWS_PER_WORKER)], dst, isem))
        for h in loads:
            h.wait()

        lacc_v[...] = jnp.zeros((LANES,), jnp.float32)

        combos = [
            (ent_e, iph), (ent_t, iph),
            (ent_e, ipt), (ent_t, ipt),
            (rel_e, ipr), (rel_t, ipr),
            (ent_e, inh), (ent_t, inh),
            (ent_e, int_), (ent_t, int_),
            (rel_e, inr), (rel_t, inr),
        ]

        rows0 = lax.iota(jnp.int32, LANES)
        UNROLL = 4

        def fire(chunk, dset, sem):
            off = chunk * CHUNK
            for a, (tbl, ib) in enumerate(combos):
                pltpu.async_copy(
                    tbl.at[ib.at[pl.ds(off, CHUNK)]], data.at[dset, a], sem)

        def drain(chunk, dset, sem):
            off = chunk * CHUNK
            for a, (tbl, ib) in enumerate(combos):
                pltpu.make_async_copy(
                    tbl.at[ib.at[pl.ds(off, CHUNK)]], data.at[dset, a],
                    sem).wait()

        def side_score(d_, a0):
            def body(fu, carry):
                (s, dd, drp, rr) = carry
                f0 = fu * UNROLL
                for u in range(UNROLL):
                    col = (rows0 + (f0 + u)) & (DIM - 1)
                    he = plsc.load_gather(d_.at[a0 + 0], [rows0, col])
                    hp = plsc.load_gather(d_.at[a0 + 1], [rows0, col])
                    te = plsc.load_gather(d_.at[a0 + 2], [rows0, col])
                    tp = plsc.load_gather(d_.at[a0 + 3], [rows0, col])
                    re = plsc.load_gather(d_.at[a0 + 4], [rows0, col])
                    rp = plsc.load_gather(d_.at[a0 + 5], [rows0, col])
                    s = s + he * hp - te * tp
                    d = he - te + re
                    dd = dd + d * d
                    drp = drp + d * rp
                    rr = rr + rp * rp
                return (s, dd, drp, rr)

            z = jnp.zeros((LANES,), jnp.float32)
            (s, dd, drp, rr) = lax.fori_loop(
                0, DIM // UNROLL, body, (z, z, z, z))
            return dd + 2.0 * s * drp + s * s * rr

        def compute(dset):
            d_ = data.at[dset]
            ps = side_score(d_, 0)
            ns = side_score(d_, 6)
            pairs = jnp.maximum(ns - ps + MARGIN, 0.0)
            lacc_v[...] = lacc_v[...] + pairs

        gsems = [gsem0, gsem1, gsem2, gsem3]
        for c in range(NSETS - 1):
            fire(c, c, gsems[c])

        @pl.loop(0, NUM_CHUNKS // NSETS)
        def _(p):
            c0 = p * NSETS
            for j in range(NSETS):
                sj = (j + NSETS - 1) % NSETS

                @pl.when(c0 + j + NSETS - 1 < NUM_CHUNKS)
                def _(c=c0 + j + NSETS - 1, sj=sj):
                    fire(c, sj, gsems[sj])

                drain(c0 + j, j, gsems[j])
                compute(j)

        pltpu.sync_copy(lacc_v, out.at[wid])

    return transd_sc


_transd_sc = _make_sc_kernel()


def kernel(pos_h, pos_t, pos_r, neg_h, neg_t, neg_r,
           ent_embeddings, rel_embeddings, ent_transfer, rel_transfer):
    partials = _transd_sc(ent_embeddings, ent_transfer, rel_embeddings,
                          rel_transfer, pos_h, pos_t, pos_r,
                          neg_h, neg_t, neg_r)
    return jnp.sum(partials)


# speedup vs baseline: 1.0229x; 1.0229x over previous
import dataclasses
import functools

import jax
import jax.numpy as jnp
from jax import lax
from jax.experimental import pallas as pl
from jax.experimental.pallas import tpu as pltpu
from jax.experimental.pallas import tpu_sc as plsc

BATCH = 16384
DIM = 128
LANES = 16
NUM_WORKERS = 32
ROWS_PER_WORKER = BATCH // NUM_WORKERS
CHUNK = 16
NUM_CHUNKS = ROWS_PER_WORKER // CHUNK
GROUPS = CHUNK // LANES
NSETS = 4
NARR = 12
MARGIN = 1.0


def _make_sc_kernel():
    mesh = plsc.VectorSubcoreMesh(core_axis_name="c", subcore_axis_name="s")
    cp = pltpu.CompilerParams()
    if "needs_layout_passes" in pltpu.CompilerParams.__dataclass_fields__:
        cp = dataclasses.replace(cp, needs_layout_passes=False)

    @functools.partial(
        pl.kernel,
        out_type=jax.ShapeDtypeStruct((NUM_WORKERS, LANES), jnp.float32),
        mesh=mesh,
        compiler_params=cp,
        scratch_types=(
            [pltpu.VMEM((ROWS_PER_WORKER,), jnp.int32)] * 6
            + [pltpu.VMEM((NSETS, NARR, CHUNK, DIM), jnp.float32)]
            + [pltpu.VMEM((LANES,), jnp.float32)]
            + [pltpu.SemaphoreType.DMA] * (NSETS + 1)
        ),
    )
    def transd_sc(ent_e, ent_t, rel_e, rel_t,
                  ph, pt, pr, nh, nt, nr,
                  out,
                  iph, ipt, ipr, inh, int_, inr,
                  data, lacc_v, isem, gsem0, gsem1, gsem2, gsem3):
        wid = lax.axis_index("s") * 2 + lax.axis_index("c")
        base0 = wid * ROWS_PER_WORKER

        idx_bufs = [iph, ipt, ipr, inh, int_, inr]
        loads = []
        for src, dst in zip([ph, pt, pr, nh, nt, nr], idx_bufs):
            loads.append(pltpu.async_copy(
                src.at[pl.ds(base0, ROWS_PER_WORKER)], dst, isem))
        for h in loads:
            h.wait()

        lacc_v[...] = jnp.zeros((LANES,), jnp.float32)

        combos = [
            (ent_e, iph), (ent_t, iph),
            (ent_e, ipt), (ent_t, ipt),
            (rel_e, ipr), (rel_t, ipr),
            (ent_e, inh), (ent_t, inh),
            (ent_e, int_), (ent_t, int_),
            (rel_e, inr), (rel_t, inr),
        ]

        rows0 = lax.iota(jnp.int32, LANES)
        UNROLL = 2

        def fire(chunk, dset, sem):
            off = chunk * CHUNK
            for a, (tbl, ib) in enumerate(combos):
                pltpu.async_copy(
                    tbl.at[ib.at[pl.ds(off, CHUNK)]], data.at[dset, a], sem)

        def drain(chunk, dset, sem):
            off = chunk * CHUNK
            for a, (tbl, ib) in enumerate(combos):
                pltpu.make_async_copy(
                    tbl.at[ib.at[pl.ds(off, CHUNK)]], data.at[dset, a],
                    sem).wait()

        def compute(dset):
            d_ = data.at[dset]
            for g in range(GROUPS):
                rows = rows0 + g * LANES

                def body(fu, carry, rows=rows):
                    (sp, ddp, drpp, rrp, sn, ddn, drpn, rrn) = carry
                    f0 = fu * UNROLL
                    for u in range(UNROLL):
                        col = (rows0 + (f0 + u)) & (DIM - 1)
                        he = plsc.load_gather(d_.at[0], [rows, col])
                        hp = plsc.load_gather(d_.at[1], [rows, col])
                        te = plsc.load_gather(d_.at[2], [rows, col])
                        tp = plsc.load_gather(d_.at[3], [rows, col])
                        re = plsc.load_gather(d_.at[4], [rows, col])
                        rp = plsc.load_gather(d_.at[5], [rows, col])
                        sp = sp + he * hp - te * tp
                        d = he - te + re
                        ddp = ddp + d * d
                        drpp = drpp + d * rp
                        rrp = rrp + rp * rp
                        he = plsc.load_gather(d_.at[6], [rows, col])
                        hp = plsc.load_gather(d_.at[7], [rows, col])
                        te = plsc.load_gather(d_.at[8], [rows, col])
                        tp = plsc.load_gather(d_.at[9], [rows, col])
                        re = plsc.load_gather(d_.at[10], [rows, col])
                        rp = plsc.load_gather(d_.at[11], [rows, col])
                        sn = sn + he * hp - te * tp
                        d = he - te + re
                        ddn = ddn + d * d
                        drpn = drpn + d * rp
                        rrn = rrn + rp * rp
                    return (sp, ddp, drpp, rrp, sn, ddn, drpn, rrn)

                z = jnp.zeros((LANES,), jnp.float32)
                (sp, ddp, drpp, rrp, sn, ddn, drpn, rrn) = lax.fori_loop(
                    0, DIM // UNROLL, body, (z, z, z, z, z, z, z, z))
                ps = ddp + 2.0 * sp * drpp + sp * sp * rrp
                ns = ddn + 2.0 * sn * drpn + sn * sn * rrn
                pairs = jnp.maximum(ns - ps + MARGIN, 0.0)
                lacc_v[...] = lacc_v[...] + pairs

        gsems = [gsem0, gsem1, gsem2, gsem3]
        for c in range(NSETS - 1):
            fire(c, c, gsems[c])

        @pl.loop(0, NUM_CHUNKS // NSETS)
        def _(p):
            c0 = p * NSETS
            for j in range(NSETS):
                sj = (j + NSETS - 1) % NSETS

                @pl.when(c0 + j + NSETS - 1 < NUM_CHUNKS)
                def _(c=c0 + j + NSETS - 1, sj=sj):
                    fire(c, sj, gsems[sj])

                drain(c0 + j, j, gsems[j])
                compute(j)

        pltpu.sync_copy(lacc_v, out.at[wid])

    return transd_sc


_transd_sc = _make_sc_kernel()


def kernel(pos_h, pos_t, pos_r, neg_h, neg_t, neg_r,
           ent_embeddings, rel_embeddings, ent_transfer, rel_transfer):
    partials = _transd_sc(ent_embeddings, ent_transfer, rel_embeddings,
                          rel_transfer, pos_h, pos_t, pos_r,
                          neg_h, neg_t, neg_r)
    return jnp.sum(partials)
